# Initial kernel scaffold; baseline (speedup 1.0000x reference)
import jax
import jax.numpy as jnp
from jax.experimental import pallas as pl


def kernel(x, edge_index, batch, W1, b1, W2, b2, Wf, bf):
    raise NotImplementedError("write your pallas kernel here")


# speedup vs baseline: 26.5769x; 26.5769x over previous
import functools

import jax
import jax.numpy as jnp
from jax import lax
from jax.experimental import pallas as pl
from jax.experimental.pallas import tpu as pltpu
from jax.experimental.pallas import tpu_sc as plsc

N = 10000
E = 320000
G = 64
D_IN = 128
D_H = 16
NCLS = 10

NC = 2
NS = 16
NW = NC * NS

NP = 10240
EPT = 10240
EPAD = EPT * NW
CHUNK = 128
RPT = EPT // CHUNK
SC_ROWS = 16
NSC = RPT // SC_ROWS
ZROWS = NP // NS

_mesh = plsc.VectorSubcoreMesh(core_axis_name="c", subcore_axis_name="s")
_sc_params = pltpu.CompilerParams(use_tc_tiling_on_sc=False)


def _hist_body(dst_hbm, zero_hbm, ones_hbm, out_hbm, dstv, rows, acc, sem):
    del sem
    c = lax.axis_index("c")
    s = lax.axis_index("s")
    pltpu.sync_copy(zero_hbm.at[pl.ds(s * ZROWS, ZROWS)],
                    acc.at[pl.ds(s * ZROWS, ZROWS)])
    pltpu.sync_copy(ones_hbm, rows)
    plsc.subcore_barrier()
    w = c * NS + s
    for i in range(NSC):
        r = w * RPT + i * SC_ROWS
        pltpu.sync_copy(dst_hbm.at[pl.ds(r, SC_ROWS)], dstv)
        for j in range(SC_ROWS):
            pltpu.sync_copy(rows, acc.at[dstv.at[j]], add=True)
    plsc.subcore_barrier()
    pltpu.sync_copy(acc.at[pl.ds(s * ZROWS, ZROWS)],
                    out_hbm.at[c, pl.ds(s * ZROWS, ZROWS)])


_sc_hist = pl.kernel(
    _hist_body,
    out_type=jax.ShapeDtypeStruct((NC, NP, D_H), jnp.float32),
    mesh=_mesh,
    compiler_params=_sc_params,
    scratch_types=[
        pltpu.VMEM((SC_ROWS, CHUNK), jnp.int32),
        pltpu.VMEM((CHUNK, D_H), jnp.float32),
        pltpu.VMEM_SHARED((NP, D_H), jnp.float32),
        pltpu.SemaphoreType.DMA,
    ],
)


def _agg_body(hp_hbm, src_hbm, dst_hbm, zero_hbm, out_hbm,
              srcv, dstv, rows, acc, sem):
    c = lax.axis_index("c")
    s = lax.axis_index("s")
    pltpu.sync_copy(zero_hbm.at[pl.ds(s * ZROWS, ZROWS)],
                    acc.at[pl.ds(s * ZROWS, ZROWS)])
    plsc.subcore_barrier()
    w = c * NS + s
    for i in range(NSC):
        r = w * RPT + i * SC_ROWS
        pltpu.sync_copy(src_hbm.at[pl.ds(r, SC_ROWS)], srcv)
        pltpu.sync_copy(dst_hbm.at[pl.ds(r, SC_ROWS)], dstv)
        for j in range(SC_ROWS):
            pltpu.async_copy(hp_hbm.at[srcv.at[j]], rows, sem).wait()
            pltpu.sync_copy(rows, acc.at[dstv.at[j]], add=True)
    plsc.subcore_barrier()
    pltpu.sync_copy(acc.at[pl.ds(s * ZROWS, ZROWS)],
                    out_hbm.at[c, pl.ds(s * ZROWS, ZROWS)])


_sc_agg = pl.kernel(
    _agg_body,
    out_type=jax.ShapeDtypeStruct((NC, NP, D_H), jnp.float32),
    mesh=_mesh,
    compiler_params=_sc_params,
    scratch_types=[
        pltpu.VMEM((SC_ROWS, CHUNK), jnp.int32),
        pltpu.VMEM((SC_ROWS, CHUNK), jnp.int32),
        pltpu.VMEM((CHUNK, D_H), jnp.float32),
        pltpu.VMEM_SHARED((NP, D_H), jnp.float32),
        pltpu.SemaphoreType.DMA,
    ],
)


_R = 1024
_GRID = NP // _R


def _tc1_body(x_ref, w1_ref, d0_ref, d1_ref, h1p_ref, dinv_ref):
    dinv = lax.rsqrt(d0_ref[...] + d1_ref[...] + 1.0)
    h1 = jnp.dot(x_ref[...], w1_ref[...], preferred_element_type=jnp.float32)
    h1p_ref[...] = h1 * dinv
    dinv_ref[...] = dinv


_tc1 = pl.pallas_call(
    _tc1_body,
    grid=(_GRID,),
    in_specs=[
        pl.BlockSpec((_R, D_IN), lambda i: (i, 0)),
        pl.BlockSpec((D_IN, D_H), lambda i: (0, 0)),
        pl.BlockSpec((_R, D_H), lambda i: (i, 0)),
        pl.BlockSpec((_R, D_H), lambda i: (i, 0)),
    ],
    out_specs=[
        pl.BlockSpec((_R, D_H), lambda i: (i, 0)),
        pl.BlockSpec((_R, D_H), lambda i: (i, 0)),
    ],
    out_shape=[
        jax.ShapeDtypeStruct((NP, D_H), jnp.float32),
        jax.ShapeDtypeStruct((NP, D_H), jnp.float32),
    ],
)


def _tc2_body(s0_ref, s1_ref, h1p_ref, dinv_ref, b1_ref, w2_ref, h2p_ref):
    dinv = dinv_ref[...]
    a = dinv * (s0_ref[...] + s1_ref[...] + h1p_ref[...]) + b1_ref[...]
    r = jnp.maximum(a, 0.0)
    h2 = jnp.dot(r, w2_ref[...], preferred_element_type=jnp.float32)
    h2p_ref[...] = h2 * dinv


_tc2 = pl.pallas_call(
    _tc2_body,
    grid=(_GRID,),
    in_specs=[
        pl.BlockSpec((_R, D_H), lambda i: (i, 0)),
        pl.BlockSpec((_R, D_H), lambda i: (i, 0)),
        pl.BlockSpec((_R, D_H), lambda i: (i, 0)),
        pl.BlockSpec((_R, D_H), lambda i: (i, 0)),
        pl.BlockSpec((1, D_H), lambda i: (0, 0)),
        pl.BlockSpec((D_H, D_H), lambda i: (0, 0)),
    ],
    out_specs=pl.BlockSpec((_R, D_H), lambda i: (i, 0)),
    out_shape=jax.ShapeDtypeStruct((NP, D_H), jnp.float32),
)


def _tc3_body(s0_ref, s1_ref, h2p_ref, dinv_ref, b2_ref, batch_ref,
              wf_ref, bf_ref, out_ref, psum_ref, cnt_ref):
    i = pl.program_id(0)

    @pl.when(i == 0)
    def _init():
        psum_ref[...] = jnp.zeros_like(psum_ref)
        cnt_ref[...] = jnp.zeros_like(cnt_ref)

    dinv = dinv_ref[...]
    out2 = dinv * (s0_ref[...] + s1_ref[...] + h2p_ref[...]) + b2_ref[...]
    b = batch_ref[...].reshape(1, _R)
    onehot = (lax.broadcasted_iota(jnp.int32, (G, _R), 0) == b
              ).astype(jnp.float32)
    psum_ref[...] += jnp.dot(onehot, out2, preferred_element_type=jnp.float32)
    cnt_ref[...] += jnp.dot(onehot, jnp.ones((_R, D_H), jnp.float32),
                            preferred_element_type=jnp.float32)

    @pl.when(i == _GRID - 1)
    def _fin():
        pooled = psum_ref[...] / jnp.maximum(cnt_ref[...], 1.0)
        out_ref[...] = jnp.dot(pooled, wf_ref[...],
                               preferred_element_type=jnp.float32) + bf_ref[...]


_tc3 = pl.pallas_call(
    _tc3_body,
    grid=(_GRID,),
    in_specs=[
        pl.BlockSpec((_R, D_H), lambda i: (i, 0)),
        pl.BlockSpec((_R, D_H), lambda i: (i, 0)),
        pl.BlockSpec((_R, D_H), lambda i: (i, 0)),
        pl.BlockSpec((_R, D_H), lambda i: (i, 0)),
        pl.BlockSpec((1, D_H), lambda i: (0, 0)),
        pl.BlockSpec((1, 1, _R), lambda i: (i, 0, 0)),
        pl.BlockSpec((D_H, NCLS), lambda i: (0, 0)),
        pl.BlockSpec((1, NCLS), lambda i: (0, 0)),
    ],
    out_specs=pl.BlockSpec((G, NCLS), lambda i: (0, 0)),
    out_shape=jax.ShapeDtypeStruct((G, NCLS), jnp.float32),
    scratch_shapes=[
        pltpu.VMEM((G, D_H), jnp.float32),
        pltpu.VMEM((G, D_H), jnp.float32),
    ],
)


def kernel(x, edge_index, batch, W1, b1, W2, b2, Wf, bf):
    pad = jnp.full((EPAD - E,), NP - 1, dtype=jnp.int32)
    src2d = jnp.concatenate([edge_index[0], pad]).reshape(-1, CHUNK)
    dst2d = jnp.concatenate([edge_index[1], pad]).reshape(-1, CHUNK)
    xp = jnp.pad(x, ((0, NP - N), (0, 0)))
    batchp = jnp.concatenate(
        [batch, jnp.full((NP - N,), G, dtype=jnp.int32)]).reshape(_GRID, 1, _R)
    zeros_np = jnp.zeros((NP, D_H), jnp.float32)
    ones_chunk = jnp.ones((CHUNK, D_H), jnp.float32)

    deg = _sc_hist(dst2d, zeros_np, ones_chunk)
    h1p, dinv = _tc1(xp, W1, deg[0], deg[1])
    s1 = _sc_agg(h1p, src2d, dst2d, zeros_np)
    h2p = _tc2(s1[0], s1[1], h1p, dinv, b1.reshape(1, D_H), W2)
    s2 = _sc_agg(h2p, src2d, dst2d, zeros_np)
    return _tc3(s2[0], s2[1], h2p, dinv, b2.reshape(1, D_H),
                batchp, Wf, bf.reshape(1, NCLS))


# speedup vs baseline: 35.5875x; 1.3390x over previous
import functools

import jax
import jax.numpy as jnp
from jax import lax
from jax.experimental import pallas as pl
from jax.experimental.pallas import tpu as pltpu
from jax.experimental.pallas import tpu_sc as plsc

N = 10000
E = 320000
G = 64
D_IN = 128
D_H = 16
NCLS = 10

NC = 2
NS = 16
NW = NC * NS

NP = 10240
EPT = 10240
EPAD = EPT * NW
CHUNK = 128
RPT = EPT // CHUNK
SC_ROWS = 16
NSC = RPT // SC_ROWS
ZROWS = NP // NS

_mesh = plsc.VectorSubcoreMesh(core_axis_name="c", subcore_axis_name="s")
_sc_params = pltpu.CompilerParams(use_tc_tiling_on_sc=False)


NBUF = 8
LAG = 4
NG = RPT // NBUF


def _hist_body(dst_hbm, zero_hbm, ones_hbm, out_hbm, dstv, rows, acc, *sems):
    c = lax.axis_index("c")
    s = lax.axis_index("s")
    pltpu.sync_copy(zero_hbm.at[pl.ds(s * ZROWS, ZROWS)],
                    acc.at[pl.ds(s * ZROWS, ZROWS)])
    pltpu.sync_copy(ones_hbm, rows)
    w = c * NS + s
    pltpu.sync_copy(dst_hbm.at[pl.ds(w * RPT, RPT)], dstv)
    plsc.subcore_barrier()

    def s_start(k, b):
        pltpu.async_copy(rows, acc.at[dstv.at[k]], sems[b], add=True)

    def s_wait(k, b):
        pltpu.make_async_copy(rows, acc.at[dstv.at[k]], sems[b]).wait()

    def group(g, _):
        for b in range(NBUF):
            k = g * NBUF + b

            @pl.when(k >= NBUF)
            def _():
                s_wait(k - NBUF, b)

            s_start(k, b)
        return _

    lax.fori_loop(0, NG, group, None, unroll=False)
    for b in range(NBUF):
        s_wait(RPT - NBUF + b, b)
    plsc.subcore_barrier()
    pltpu.sync_copy(acc.at[pl.ds(s * ZROWS, ZROWS)],
                    out_hbm.at[c, pl.ds(s * ZROWS, ZROWS)])


_sc_hist = pl.kernel(
    _hist_body,
    out_type=jax.ShapeDtypeStruct((NC, NP, D_H), jnp.float32),
    mesh=_mesh,
    compiler_params=_sc_params,
    scratch_types=[
        pltpu.VMEM((RPT, CHUNK), jnp.int32),
        pltpu.VMEM((CHUNK, D_H), jnp.float32),
        pltpu.VMEM_SHARED((NP, D_H), jnp.float32),
    ] + [pltpu.SemaphoreType.DMA] * NBUF,
)


def _agg_body(hp_hbm, src_hbm, dst_hbm, zero_hbm, out_hbm,
              srcv, dstv, rows, acc, *sems):
    gsem = sems[:NBUF]
    ssem = sems[NBUF:]
    c = lax.axis_index("c")
    s = lax.axis_index("s")
    pltpu.sync_copy(zero_hbm.at[pl.ds(s * ZROWS, ZROWS)],
                    acc.at[pl.ds(s * ZROWS, ZROWS)])
    w = c * NS + s
    pltpu.sync_copy(src_hbm.at[pl.ds(w * RPT, RPT)], srcv)
    pltpu.sync_copy(dst_hbm.at[pl.ds(w * RPT, RPT)], dstv)
    plsc.subcore_barrier()

    def g_start(k, b):
        pltpu.async_copy(hp_hbm.at[srcv.at[k]], rows.at[b], gsem[b])

    def g_wait(k, b):
        pltpu.make_async_copy(hp_hbm.at[srcv.at[k]], rows.at[b],
                              gsem[b]).wait()

    def s_start(k, b):
        pltpu.async_copy(rows.at[b], acc.at[dstv.at[k]], ssem[b], add=True)

    def s_wait(k, b):
        pltpu.make_async_copy(rows.at[b], acc.at[dstv.at[k]],
                              ssem[b]).wait()

    for b in range(LAG):
        g_start(b, b)

    def group(g, _):
        for b in range(NBUF):
            k = g * NBUF + b
            b2 = (b + LAG) % NBUF
            g_wait(k, b)
            s_start(k, b)

            @pl.when(k >= NBUF - LAG)
            def _():
                s_wait(k + LAG - NBUF, b2)

            @pl.when(k + LAG < RPT)
            def _():
                g_start(k + LAG, b2)

        return _

    lax.fori_loop(0, NG, group, None, unroll=False)
    for j in range(NBUF - LAG):
        k = RPT - (NBUF - LAG) + j
        s_wait(k, k % NBUF)
    plsc.subcore_barrier()
    pltpu.sync_copy(acc.at[pl.ds(s * ZROWS, ZROWS)],
                    out_hbm.at[c, pl.ds(s * ZROWS, ZROWS)])


_sc_agg = pl.kernel(
    _agg_body,
    out_type=jax.ShapeDtypeStruct((NC, NP, D_H), jnp.float32),
    mesh=_mesh,
    compiler_params=_sc_params,
    scratch_types=[
        pltpu.VMEM((RPT, CHUNK), jnp.int32),
        pltpu.VMEM((RPT, CHUNK), jnp.int32),
        pltpu.VMEM((NBUF, CHUNK, D_H), jnp.float32),
        pltpu.VMEM_SHARED((NP, D_H), jnp.float32),
    ] + [pltpu.SemaphoreType.DMA] * (2 * NBUF),
)


_R = 1024
_GRID = NP // _R


def _tc1_body(x_ref, w1_ref, d0_ref, d1_ref, h1p_ref, dinv_ref):
    dinv = lax.rsqrt(d0_ref[...] + d1_ref[...] + 1.0)
    h1 = jnp.dot(x_ref[...], w1_ref[...], preferred_element_type=jnp.float32)
    h1p_ref[...] = h1 * dinv
    dinv_ref[...] = dinv


_tc1 = pl.pallas_call(
    _tc1_body,
    grid=(_GRID,),
    in_specs=[
        pl.BlockSpec((_R, D_IN), lambda i: (i, 0)),
        pl.BlockSpec((D_IN, D_H), lambda i: (0, 0)),
        pl.BlockSpec((_R, D_H), lambda i: (i, 0)),
        pl.BlockSpec((_R, D_H), lambda i: (i, 0)),
    ],
    out_specs=[
        pl.BlockSpec((_R, D_H), lambda i: (i, 0)),
        pl.BlockSpec((_R, D_H), lambda i: (i, 0)),
    ],
    out_shape=[
        jax.ShapeDtypeStruct((NP, D_H), jnp.float32),
        jax.ShapeDtypeStruct((NP, D_H), jnp.float32),
    ],
)


def _tc2_body(s0_ref, s1_ref, h1p_ref, dinv_ref, b1_ref, w2_ref, h2p_ref):
    dinv = dinv_ref[...]
    a = dinv * (s0_ref[...] + s1_ref[...] + h1p_ref[...]) + b1_ref[...]
    r = jnp.maximum(a, 0.0)
    h2 = jnp.dot(r, w2_ref[...], preferred_element_type=jnp.float32)
    h2p_ref[...] = h2 * dinv


_tc2 = pl.pallas_call(
    _tc2_body,
    grid=(_GRID,),
    in_specs=[
        pl.BlockSpec((_R, D_H), lambda i: (i, 0)),
        pl.BlockSpec((_R, D_H), lambda i: (i, 0)),
        pl.BlockSpec((_R, D_H), lambda i: (i, 0)),
        pl.BlockSpec((_R, D_H), lambda i: (i, 0)),
        pl.BlockSpec((1, D_H), lambda i: (0, 0)),
        pl.BlockSpec((D_H, D_H), lambda i: (0, 0)),
    ],
    out_specs=pl.BlockSpec((_R, D_H), lambda i: (i, 0)),
    out_shape=jax.ShapeDtypeStruct((NP, D_H), jnp.float32),
)


def _tc3_body(s0_ref, s1_ref, h2p_ref, dinv_ref, b2_ref, batch_ref,
              wf_ref, bf_ref, out_ref, psum_ref, cnt_ref):
    i = pl.program_id(0)

    @pl.when(i == 0)
    def _init():
        psum_ref[...] = jnp.zeros_like(psum_ref)
        cnt_ref[...] = jnp.zeros_like(cnt_ref)

    dinv = dinv_ref[...]
    out2 = dinv * (s0_ref[...] + s1_ref[...] + h2p_ref[...]) + b2_ref[...]
    b = batch_ref[...].reshape(1, _R)
    onehot = (lax.broadcasted_iota(jnp.int32, (G, _R), 0) == b
              ).astype(jnp.float32)
    psum_ref[...] += jnp.dot(onehot, out2, preferred_element_type=jnp.float32)
    cnt_ref[...] += jnp.dot(onehot, jnp.ones((_R, D_H), jnp.float32),
                            preferred_element_type=jnp.float32)

    @pl.when(i == _GRID - 1)
    def _fin():
        pooled = psum_ref[...] / jnp.maximum(cnt_ref[...], 1.0)
        out_ref[...] = jnp.dot(pooled, wf_ref[...],
                               preferred_element_type=jnp.float32) + bf_ref[...]


_tc3 = pl.pallas_call(
    _tc3_body,
    grid=(_GRID,),
    in_specs=[
        pl.BlockSpec((_R, D_H), lambda i: (i, 0)),
        pl.BlockSpec((_R, D_H), lambda i: (i, 0)),
        pl.BlockSpec((_R, D_H), lambda i: (i, 0)),
        pl.BlockSpec((_R, D_H), lambda i: (i, 0)),
        pl.BlockSpec((1, D_H), lambda i: (0, 0)),
        pl.BlockSpec((1, 1, _R), lambda i: (i, 0, 0)),
        pl.BlockSpec((D_H, NCLS), lambda i: (0, 0)),
        pl.BlockSpec((1, NCLS), lambda i: (0, 0)),
    ],
    out_specs=pl.BlockSpec((G, NCLS), lambda i: (0, 0)),
    out_shape=jax.ShapeDtypeStruct((G, NCLS), jnp.float32),
    scratch_shapes=[
        pltpu.VMEM((G, D_H), jnp.float32),
        pltpu.VMEM((G, D_H), jnp.float32),
    ],
)


def kernel(x, edge_index, batch, W1, b1, W2, b2, Wf, bf):
    pad = jnp.full((EPAD - E,), NP - 1, dtype=jnp.int32)
    src2d = jnp.concatenate([edge_index[0], pad]).reshape(-1, CHUNK)
    dst2d = jnp.concatenate([edge_index[1], pad]).reshape(-1, CHUNK)
    xp = jnp.pad(x, ((0, NP - N), (0, 0)))
    batchp = jnp.concatenate(
        [batch, jnp.full((NP - N,), G, dtype=jnp.int32)]).reshape(_GRID, 1, _R)
    zeros_np = jnp.zeros((NP, D_H), jnp.float32)
    ones_chunk = jnp.ones((CHUNK, D_H), jnp.float32)

    deg = _sc_hist(dst2d, zeros_np, ones_chunk)
    h1p, dinv = _tc1(xp, W1, deg[0], deg[1])
    s1 = _sc_agg(h1p, src2d, dst2d, zeros_np)
    h2p = _tc2(s1[0], s1[1], h1p, dinv, b1.reshape(1, D_H), W2)
    s2 = _sc_agg(h2p, src2d, dst2d, zeros_np)
    return _tc3(s2[0], s2[1], h2p, dinv, b2.reshape(1, D_H),
                batchp, Wf, bf.reshape(1, NCLS))


# speedup vs baseline: 38.2153x; 1.0738x over previous
import functools

import jax
import jax.numpy as jnp
from jax import lax
from jax.experimental import pallas as pl
from jax.experimental.pallas import tpu as pltpu
from jax.experimental.pallas import tpu_sc as plsc

N = 10000
E = 320000
G = 64
D_IN = 128
D_H = 16
NCLS = 10

NC = 2
NS = 16
NW = NC * NS

NP = 10240
EPT = 10240
EPAD = EPT * NW
CHUNK = 128
RPT = EPT // CHUNK
SC_ROWS = 16
NSC = RPT // SC_ROWS
ZROWS = NP // NS

_mesh = plsc.VectorSubcoreMesh(core_axis_name="c", subcore_axis_name="s")
_sc_params = pltpu.CompilerParams(use_tc_tiling_on_sc=False)


NBUF = 8
LAG = 4
NG = RPT // NBUF


def _hist_body(dst_hbm, zero_hbm, ones_hbm, out_hbm, dstv, rows, acc, *sems):
    c = lax.axis_index("c")
    s = lax.axis_index("s")
    pltpu.sync_copy(zero_hbm.at[pl.ds(s * ZROWS, ZROWS)],
                    acc.at[pl.ds(s * ZROWS, ZROWS)])
    pltpu.sync_copy(ones_hbm, rows)
    w = c * NS + s
    pltpu.sync_copy(dst_hbm.at[pl.ds(w * RPT, RPT)], dstv)
    plsc.subcore_barrier()

    def s_start(k, b):
        pltpu.async_copy(rows, acc.at[dstv.at[k]], sems[b], add=True)

    def s_wait(k, b):
        pltpu.make_async_copy(rows, acc.at[dstv.at[k]], sems[b]).wait()

    def group(g, _):
        for b in range(NBUF):
            k = g * NBUF + b

            @pl.when(k >= NBUF)
            def _():
                s_wait(k - NBUF, b)

            s_start(k, b)
        return _

    lax.fori_loop(0, NG, group, None, unroll=False)
    for b in range(NBUF):
        s_wait(RPT - NBUF + b, b)
    plsc.subcore_barrier()
    pltpu.sync_copy(acc.at[pl.ds(s * ZROWS, ZROWS)],
                    out_hbm.at[c, pl.ds(s * ZROWS, ZROWS)])


_sc_hist = pl.kernel(
    _hist_body,
    out_type=jax.ShapeDtypeStruct((NC, NP, D_H), jnp.float32),
    mesh=_mesh,
    compiler_params=_sc_params,
    scratch_types=[
        pltpu.VMEM((RPT, CHUNK), jnp.int32),
        pltpu.VMEM((CHUNK, D_H), jnp.float32),
        pltpu.VMEM_SHARED((NP, D_H), jnp.float32),
    ] + [pltpu.SemaphoreType.DMA] * NBUF,
)


def _agg_body(hp_hbm, src_hbm, dst_hbm, zero_hbm, out_hbm,
              srcv, dstv, rows, acc, *sems):
    gsem = sems[:NBUF]
    ssem = sems[NBUF:]
    c = lax.axis_index("c")
    s = lax.axis_index("s")
    pltpu.sync_copy(zero_hbm.at[pl.ds(s * ZROWS, ZROWS)],
                    acc.at[pl.ds(s * ZROWS, ZROWS)])
    w = c * NS + s
    pltpu.sync_copy(src_hbm.at[pl.ds(w * RPT, RPT)], srcv)
    pltpu.sync_copy(dst_hbm.at[pl.ds(w * RPT, RPT)], dstv)
    plsc.subcore_barrier()

    def g_start(k, b):
        pltpu.async_copy(hp_hbm.at[srcv.at[k]], rows.at[b], gsem[b])

    def g_wait(k, b):
        pltpu.make_async_copy(hp_hbm.at[srcv.at[k]], rows.at[b],
                              gsem[b]).wait()

    def s_start(k, b):
        pltpu.async_copy(rows.at[b], acc.at[dstv.at[k]], ssem[b], add=True)

    def s_wait(k, b):
        pltpu.make_async_copy(rows.at[b], acc.at[dstv.at[k]],
                              ssem[b]).wait()

    for b in range(LAG):
        g_start(b, b)

    def group(g, _):
        for b in range(NBUF):
            k = g * NBUF + b
            b2 = (b + LAG) % NBUF
            g_wait(k, b)
            s_start(k, b)

            @pl.when(k >= NBUF - LAG)
            def _():
                s_wait(k + LAG - NBUF, b2)

            @pl.when(k + LAG < RPT)
            def _():
                g_start(k + LAG, b2)

        return _

    lax.fori_loop(0, NG, group, None, unroll=False)
    for j in range(NBUF - LAG):
        k = RPT - (NBUF - LAG) + j
        s_wait(k, k % NBUF)
    plsc.subcore_barrier()
    pltpu.sync_copy(acc.at[pl.ds(s * ZROWS, ZROWS)],
                    out_hbm.at[c, pl.ds(s * ZROWS, ZROWS)])


_sc_agg = pl.kernel(
    _agg_body,
    out_type=jax.ShapeDtypeStruct((NC, NP, D_H), jnp.float32),
    mesh=_mesh,
    compiler_params=_sc_params,
    scratch_types=[
        pltpu.VMEM((RPT, CHUNK), jnp.int32),
        pltpu.VMEM((RPT, CHUNK), jnp.int32),
        pltpu.VMEM((NBUF, CHUNK, D_H), jnp.float32),
        pltpu.VMEM_SHARED((NP, D_H), jnp.float32),
    ] + [pltpu.SemaphoreType.DMA] * (2 * NBUF),
)


_R = 1024
_GRID = NP // _R


def _tc1_body(x_ref, w1_ref, d0_ref, d1_ref, h1p_ref, dinv_ref):
    dinv = lax.rsqrt(d0_ref[0] + d1_ref[0] + 1.0)
    h1 = jnp.dot(x_ref[...], w1_ref[...], preferred_element_type=jnp.float32)
    h1p_ref[...] = h1 * dinv
    dinv_ref[...] = dinv


_tc1 = pl.pallas_call(
    _tc1_body,
    grid=(_GRID,),
    in_specs=[
        pl.BlockSpec((_R, D_IN), lambda i: (i, 0)),
        pl.BlockSpec((D_IN, D_H), lambda i: (0, 0)),
        pl.BlockSpec((1, _R, D_H), lambda i: (0, i, 0)),
        pl.BlockSpec((1, _R, D_H), lambda i: (1, i, 0)),
    ],
    out_specs=[
        pl.BlockSpec((_R, D_H), lambda i: (i, 0)),
        pl.BlockSpec((_R, D_H), lambda i: (i, 0)),
    ],
    out_shape=[
        jax.ShapeDtypeStruct((NP, D_H), jnp.float32),
        jax.ShapeDtypeStruct((NP, D_H), jnp.float32),
    ],
)


def _tc2_body(s0_ref, s1_ref, h1p_ref, dinv_ref, b1_ref, w2_ref, h2p_ref):
    dinv = dinv_ref[...]
    a = dinv * (s0_ref[0] + s1_ref[0] + h1p_ref[...]) + b1_ref[...]
    r = jnp.maximum(a, 0.0)
    h2 = jnp.dot(r, w2_ref[...], preferred_element_type=jnp.float32)
    h2p_ref[...] = h2 * dinv


_tc2 = pl.pallas_call(
    _tc2_body,
    grid=(_GRID,),
    in_specs=[
        pl.BlockSpec((1, _R, D_H), lambda i: (0, i, 0)),
        pl.BlockSpec((1, _R, D_H), lambda i: (1, i, 0)),
        pl.BlockSpec((_R, D_H), lambda i: (i, 0)),
        pl.BlockSpec((_R, D_H), lambda i: (i, 0)),
        pl.BlockSpec((1, D_H), lambda i: (0, 0)),
        pl.BlockSpec((D_H, D_H), lambda i: (0, 0)),
    ],
    out_specs=pl.BlockSpec((_R, D_H), lambda i: (i, 0)),
    out_shape=jax.ShapeDtypeStruct((NP, D_H), jnp.float32),
)


def _tc3_body(s0_ref, s1_ref, h2p_ref, dinv_ref, b2_ref, batch_ref,
              wf_ref, bf_ref, out_ref, psum_ref, cnt_ref):
    i = pl.program_id(0)

    @pl.when(i == 0)
    def _init():
        psum_ref[...] = jnp.zeros_like(psum_ref)
        cnt_ref[...] = jnp.zeros_like(cnt_ref)

    dinv = dinv_ref[...]
    out2 = dinv * (s0_ref[0] + s1_ref[0] + h2p_ref[...]) + b2_ref[...]
    b = batch_ref[...].reshape(1, _R)
    valid = (batch_ref[...].reshape(_R, 1) < G)
    out2 = jnp.where(valid, out2, 0.0)
    onehot = (lax.broadcasted_iota(jnp.int32, (G, _R), 0) == b
              ).astype(jnp.float32)
    psum_ref[...] += jnp.dot(onehot, out2, preferred_element_type=jnp.float32)
    cnt_ref[...] += jnp.dot(onehot, jnp.ones((_R, D_H), jnp.float32),
                            preferred_element_type=jnp.float32)

    @pl.when(i == _GRID - 1)
    def _fin():
        pooled = psum_ref[...] / jnp.maximum(cnt_ref[...], 1.0)
        out_ref[...] = jnp.dot(pooled, wf_ref[...],
                               preferred_element_type=jnp.float32) + bf_ref[...]


_tc3 = pl.pallas_call(
    _tc3_body,
    grid=(_GRID,),
    in_specs=[
        pl.BlockSpec((1, _R, D_H), lambda i: (0, i, 0)),
        pl.BlockSpec((1, _R, D_H), lambda i: (1, i, 0)),
        pl.BlockSpec((_R, D_H), lambda i: (i, 0)),
        pl.BlockSpec((_R, D_H), lambda i: (i, 0)),
        pl.BlockSpec((1, D_H), lambda i: (0, 0)),
        pl.BlockSpec((1, 1, _R), lambda i: (i, 0, 0)),
        pl.BlockSpec((D_H, NCLS), lambda i: (0, 0)),
        pl.BlockSpec((1, NCLS), lambda i: (0, 0)),
    ],
    out_specs=pl.BlockSpec((G, NCLS), lambda i: (0, 0)),
    out_shape=jax.ShapeDtypeStruct((G, NCLS), jnp.float32),
    scratch_shapes=[
        pltpu.VMEM((G, D_H), jnp.float32),
        pltpu.VMEM((G, D_H), jnp.float32),
    ],
)


def kernel(x, edge_index, batch, W1, b1, W2, b2, Wf, bf):
    pad = jnp.full((EPAD - E,), NP - 1, dtype=jnp.int32)
    src2d = jnp.concatenate([edge_index[0], pad]).reshape(-1, CHUNK)
    dst2d = jnp.concatenate([edge_index[1], pad]).reshape(-1, CHUNK)
    batchp = jnp.concatenate(
        [batch, jnp.full((NP - N,), G, dtype=jnp.int32)]).reshape(_GRID, 1, _R)
    zeros_np = jnp.zeros((NP, D_H), jnp.float32)
    ones_chunk = jnp.ones((CHUNK, D_H), jnp.float32)

    xp = jnp.pad(x, ((0, NP - N), (0, 0)))
    deg = _sc_hist(dst2d, zeros_np, ones_chunk)
    h1p, dinv = _tc1(xp, W1, deg, deg)
    s1 = _sc_agg(h1p, src2d, dst2d, zeros_np)
    h2p = _tc2(s1, s1, h1p, dinv, b1.reshape(1, D_H), W2)
    s2 = _sc_agg(h2p, src2d, dst2d, zeros_np)
    return _tc3(s2, s2, h2p, dinv, b2.reshape(1, D_H),
                batchp, Wf, bf.reshape(1, NCLS))


# speedup vs baseline: 38.2768x; 1.0016x over previous
import functools

import jax
import jax.numpy as jnp
from jax import lax
from jax.experimental import pallas as pl
from jax.experimental.pallas import tpu as pltpu
from jax.experimental.pallas import tpu_sc as plsc

N = 10000
E = 320000
G = 64
D_IN = 128
D_H = 16
NCLS = 10

NC = 2
NS = 16
NW = NC * NS

NP = 10240
EPT = 10240
EPAD = EPT * NW
CHUNK = 128
RPT = EPT // CHUNK
SC_ROWS = 16
NSC = RPT // SC_ROWS
ZROWS = NP // NS

_mesh = plsc.VectorSubcoreMesh(core_axis_name="c", subcore_axis_name="s")
_sc_params = pltpu.CompilerParams(use_tc_tiling_on_sc=False)


NBUF = 8
LAG = 4
NG = RPT // NBUF


def _hist_body(dst_hbm, zero_hbm, ones_hbm, out_hbm, dstv, rows, acc, *sems):
    c = lax.axis_index("c")
    s = lax.axis_index("s")
    pltpu.sync_copy(zero_hbm.at[pl.ds(s * ZROWS, ZROWS)],
                    acc.at[pl.ds(s * ZROWS, ZROWS)])
    pltpu.sync_copy(ones_hbm, rows)
    w = c * NS + s
    pltpu.sync_copy(dst_hbm.at[pl.ds(w * RPT, RPT)], dstv)
    plsc.subcore_barrier()

    def s_start(k, b):
        pltpu.async_copy(rows, acc.at[dstv.at[k]], sems[b], add=True)

    def s_wait(k, b):
        pltpu.make_async_copy(rows, acc.at[dstv.at[k]], sems[b]).wait()

    def group(g, _):
        for b in range(NBUF):
            k = g * NBUF + b

            @pl.when(k >= NBUF)
            def _():
                s_wait(k - NBUF, b)

            s_start(k, b)
        return _

    lax.fori_loop(0, NG, group, None, unroll=False)
    for b in range(NBUF):
        s_wait(RPT - NBUF + b, b)
    plsc.subcore_barrier()
    pltpu.sync_copy(acc.at[pl.ds(s * ZROWS, ZROWS)],
                    out_hbm.at[c, pl.ds(s * ZROWS, ZROWS)])


_sc_hist = pl.kernel(
    _hist_body,
    out_type=jax.ShapeDtypeStruct((NC, NP, D_H), jnp.float32),
    mesh=_mesh,
    compiler_params=_sc_params,
    scratch_types=[
        pltpu.VMEM((RPT, CHUNK), jnp.int32),
        pltpu.VMEM((CHUNK, D_H), jnp.float32),
        pltpu.VMEM_SHARED((NP, D_H), jnp.float32),
    ] + [pltpu.SemaphoreType.DMA] * NBUF,
)


def _agg_body(hp_hbm, src_hbm, dst_hbm, zero_hbm, out_hbm,
              srcv, dstv, rows, acc, *sems):
    gsem = sems[:NBUF]
    ssem = sems[NBUF:]
    c = lax.axis_index("c")
    s = lax.axis_index("s")
    pltpu.sync_copy(zero_hbm.at[pl.ds(s * ZROWS, ZROWS)],
                    acc.at[pl.ds(s * ZROWS, ZROWS)])
    w = c * NS + s
    pltpu.sync_copy(src_hbm.at[pl.ds(w * RPT, RPT)], srcv)
    pltpu.sync_copy(dst_hbm.at[pl.ds(w * RPT, RPT)], dstv)
    plsc.subcore_barrier()

    def g_start(k, b):
        pltpu.async_copy(hp_hbm.at[srcv.at[k]], rows.at[b], gsem[b])

    def g_wait(k, b):
        pltpu.make_async_copy(hp_hbm.at[srcv.at[k]], rows.at[b],
                              gsem[b]).wait()

    def s_start(k, b):
        pltpu.async_copy(rows.at[b], acc.at[dstv.at[k]], ssem[b], add=True)

    def s_wait(k, b):
        pltpu.make_async_copy(rows.at[b], acc.at[dstv.at[k]],
                              ssem[b]).wait()

    for b in range(LAG):
        g_start(b, b)

    def group(g, _):
        for b in range(NBUF):
            k = g * NBUF + b
            b2 = (b + LAG) % NBUF
            g_wait(k, b)
            s_start(k, b)

            @pl.when(k >= NBUF - LAG)
            def _():
                s_wait(k + LAG - NBUF, b2)

            @pl.when(k + LAG < RPT)
            def _():
                g_start(k + LAG, b2)

        return _

    lax.fori_loop(0, NG, group, None, unroll=False)
    for j in range(NBUF - LAG):
        k = RPT - (NBUF - LAG) + j
        s_wait(k, k % NBUF)
    plsc.subcore_barrier()
    pltpu.sync_copy(acc.at[pl.ds(s * ZROWS, ZROWS)],
                    out_hbm.at[c, pl.ds(s * ZROWS, ZROWS)])


_sc_agg = pl.kernel(
    _agg_body,
    out_type=jax.ShapeDtypeStruct((NC, NP, D_H), jnp.float32),
    mesh=_mesh,
    compiler_params=_sc_params,
    scratch_types=[
        pltpu.VMEM((RPT, CHUNK), jnp.int32),
        pltpu.VMEM((RPT, CHUNK), jnp.int32),
        pltpu.VMEM((NBUF, CHUNK, D_H), jnp.float32),
        pltpu.VMEM_SHARED((NP, D_H), jnp.float32),
    ] + [pltpu.SemaphoreType.DMA] * (2 * NBUF),
)


_R = 1024
_GRID = NP // _R


def _tca_body(x_ref, w1_ref, h1_ref):
    h1_ref[...] = jnp.dot(x_ref[...], w1_ref[...],
                          preferred_element_type=jnp.float32)


_tca = pl.pallas_call(
    _tca_body,
    grid=(_GRID,),
    in_specs=[
        pl.BlockSpec((_R, D_IN), lambda i: (i, 0)),
        pl.BlockSpec((D_IN, D_H), lambda i: (0, 0)),
    ],
    out_specs=pl.BlockSpec((_R, D_H), lambda i: (i, 0)),
    out_shape=jax.ShapeDtypeStruct((NP, D_H), jnp.float32),
)


def _tcb_body(h1_ref, d0_ref, d1_ref, h1p_ref, dinv_ref):
    dinv = lax.rsqrt(d0_ref[0] + d1_ref[0] + 1.0)
    h1p_ref[...] = h1_ref[...] * dinv
    dinv_ref[...] = dinv


_tcb = pl.pallas_call(
    _tcb_body,
    grid=(_GRID,),
    in_specs=[
        pl.BlockSpec((_R, D_H), lambda i: (i, 0)),
        pl.BlockSpec((1, _R, D_H), lambda i: (0, i, 0)),
        pl.BlockSpec((1, _R, D_H), lambda i: (1, i, 0)),
    ],
    out_specs=[
        pl.BlockSpec((_R, D_H), lambda i: (i, 0)),
        pl.BlockSpec((_R, D_H), lambda i: (i, 0)),
    ],
    out_shape=[
        jax.ShapeDtypeStruct((NP, D_H), jnp.float32),
        jax.ShapeDtypeStruct((NP, D_H), jnp.float32),
    ],
)


def _tc2_body(s0_ref, s1_ref, h1p_ref, dinv_ref, b1_ref, w2_ref, h2p_ref):
    dinv = dinv_ref[...]
    a = dinv * (s0_ref[0] + s1_ref[0] + h1p_ref[...]) + b1_ref[...]
    r = jnp.maximum(a, 0.0)
    h2 = jnp.dot(r, w2_ref[...], preferred_element_type=jnp.float32)
    h2p_ref[...] = h2 * dinv


_tc2 = pl.pallas_call(
    _tc2_body,
    grid=(_GRID,),
    in_specs=[
        pl.BlockSpec((1, _R, D_H), lambda i: (0, i, 0)),
        pl.BlockSpec((1, _R, D_H), lambda i: (1, i, 0)),
        pl.BlockSpec((_R, D_H), lambda i: (i, 0)),
        pl.BlockSpec((_R, D_H), lambda i: (i, 0)),
        pl.BlockSpec((1, D_H), lambda i: (0, 0)),
        pl.BlockSpec((D_H, D_H), lambda i: (0, 0)),
    ],
    out_specs=pl.BlockSpec((_R, D_H), lambda i: (i, 0)),
    out_shape=jax.ShapeDtypeStruct((NP, D_H), jnp.float32),
)


def _tc3_body(s0_ref, s1_ref, h2p_ref, dinv_ref, b2_ref, batch_ref,
              wf_ref, bf_ref, out_ref, psum_ref, cnt_ref):
    i = pl.program_id(0)

    @pl.when(i == 0)
    def _init():
        psum_ref[...] = jnp.zeros_like(psum_ref)
        cnt_ref[...] = jnp.zeros_like(cnt_ref)

    dinv = dinv_ref[...]
    out2 = dinv * (s0_ref[0] + s1_ref[0] + h2p_ref[...]) + b2_ref[...]
    b = batch_ref[...].reshape(1, _R)
    valid = (batch_ref[...].reshape(_R, 1) < G)
    out2 = jnp.where(valid, out2, 0.0)
    onehot = (lax.broadcasted_iota(jnp.int32, (G, _R), 0) == b
              ).astype(jnp.float32)
    psum_ref[...] += jnp.dot(onehot, out2, preferred_element_type=jnp.float32)
    cnt_ref[...] += jnp.dot(onehot, jnp.ones((_R, D_H), jnp.float32),
                            preferred_element_type=jnp.float32)

    @pl.when(i == _GRID - 1)
    def _fin():
        pooled = psum_ref[...] / jnp.maximum(cnt_ref[...], 1.0)
        out_ref[...] = jnp.dot(pooled, wf_ref[...],
                               preferred_element_type=jnp.float32) + bf_ref[...]


_tc3 = pl.pallas_call(
    _tc3_body,
    grid=(_GRID,),
    in_specs=[
        pl.BlockSpec((1, _R, D_H), lambda i: (0, i, 0)),
        pl.BlockSpec((1, _R, D_H), lambda i: (1, i, 0)),
        pl.BlockSpec((_R, D_H), lambda i: (i, 0)),
        pl.BlockSpec((_R, D_H), lambda i: (i, 0)),
        pl.BlockSpec((1, D_H), lambda i: (0, 0)),
        pl.BlockSpec((1, 1, _R), lambda i: (i, 0, 0)),
        pl.BlockSpec((D_H, NCLS), lambda i: (0, 0)),
        pl.BlockSpec((1, NCLS), lambda i: (0, 0)),
    ],
    out_specs=pl.BlockSpec((G, NCLS), lambda i: (0, 0)),
    out_shape=jax.ShapeDtypeStruct((G, NCLS), jnp.float32),
    scratch_shapes=[
        pltpu.VMEM((G, D_H), jnp.float32),
        pltpu.VMEM((G, D_H), jnp.float32),
    ],
)


def kernel(x, edge_index, batch, W1, b1, W2, b2, Wf, bf):
    pad = jnp.full((EPAD - E,), NP - 1, dtype=jnp.int32)
    src2d = jnp.concatenate([edge_index[0], pad]).reshape(-1, CHUNK)
    dst2d = jnp.concatenate([edge_index[1], pad]).reshape(-1, CHUNK)
    batchp = jnp.concatenate(
        [batch, jnp.full((NP - N,), G, dtype=jnp.int32)]).reshape(_GRID, 1, _R)
    zeros_np = jnp.zeros((NP, D_H), jnp.float32)
    ones_chunk = jnp.ones((CHUNK, D_H), jnp.float32)

    xp = jnp.pad(x, ((0, NP - N), (0, 0)))
    h1 = _tca(xp, W1)
    deg = _sc_hist(dst2d, zeros_np, ones_chunk)
    h1p, dinv = _tcb(h1, deg, deg)
    s1 = _sc_agg(h1p, src2d, dst2d, zeros_np)
    h2p = _tc2(s1, s1, h1p, dinv, b1.reshape(1, D_H), W2)
    s2 = _sc_agg(h2p, src2d, dst2d, zeros_np)
    return _tc3(s2, s2, h2p, dinv, b2.reshape(1, D_H),
                batchp, Wf, bf.reshape(1, NCLS))


# speedup vs baseline: 53.4800x; 1.3972x over previous
import functools

import jax
import jax.numpy as jnp
from jax import lax
from jax.experimental import pallas as pl
from jax.experimental.pallas import tpu as pltpu
from jax.experimental.pallas import tpu_sc as plsc

N = 10000
E = 320000
G = 64
D_IN = 128
D_H = 16
NCLS = 10

NC = 2
NS = 16
NW = NC * NS

NP = 10240
EPT = 10240
EPAD = EPT * NW
CHUNK = 128
RPT = EPT // CHUNK
SC_ROWS = 16
NSC = RPT // SC_ROWS
ZROWS = NP // NS

_mesh = plsc.VectorSubcoreMesh(core_axis_name="c", subcore_axis_name="s")
_sc_params = pltpu.CompilerParams(use_tc_tiling_on_sc=False)


NBUF = 8
LAG = 4
NG = RPT // NBUF


def _hist_body(dst_hbm, zero_hbm, ones_hbm, out_hbm, dstv, rows, acc, *sems):
    c = lax.axis_index("c")
    s = lax.axis_index("s")
    pltpu.sync_copy(zero_hbm.at[pl.ds(s * ZROWS, ZROWS)],
                    acc.at[pl.ds(s * ZROWS, ZROWS)])
    pltpu.sync_copy(ones_hbm, rows)
    w = c * NS + s
    pltpu.sync_copy(dst_hbm.at[pl.ds(w * RPT, RPT)], dstv)
    plsc.subcore_barrier()

    def s_start(k, b):
        pltpu.async_copy(rows, acc.at[dstv.at[k]], sems[b], add=True)

    def s_wait(k, b):
        pltpu.make_async_copy(rows, acc.at[dstv.at[k]], sems[b]).wait()

    def group(g, _):
        for b in range(NBUF):
            k = g * NBUF + b

            @pl.when(k >= NBUF)
            def _():
                s_wait(k - NBUF, b)

            s_start(k, b)
        return _

    lax.fori_loop(0, NG, group, None, unroll=False)
    for b in range(NBUF):
        s_wait(RPT - NBUF + b, b)
    plsc.subcore_barrier()
    pltpu.sync_copy(acc.at[pl.ds(s * ZROWS, ZROWS)],
                    out_hbm.at[c, pl.ds(s * ZROWS, ZROWS)])


_sc_hist = pl.kernel(
    _hist_body,
    out_type=jax.ShapeDtypeStruct((NC, NP, D_H), jnp.float32),
    mesh=_mesh,
    compiler_params=_sc_params,
    scratch_types=[
        pltpu.VMEM((RPT, CHUNK), jnp.int32),
        pltpu.VMEM((CHUNK, D_H), jnp.float32),
        pltpu.VMEM_SHARED((NP, D_H), jnp.float32),
    ] + [pltpu.SemaphoreType.DMA] * NBUF,
)


def _agg_body(hp_hbm, src_hbm, dst_hbm, zero_hbm, out_hbm,
              srcv, dstv, rows, acc, hps, *sems):
    gsem = sems[:NBUF]
    ssem = sems[NBUF:]
    c = lax.axis_index("c")
    s = lax.axis_index("s")
    pltpu.sync_copy(zero_hbm.at[pl.ds(s * ZROWS, ZROWS)],
                    acc.at[pl.ds(s * ZROWS, ZROWS)])
    pltpu.sync_copy(hp_hbm.at[pl.ds(s * ZROWS, ZROWS)],
                    hps.at[pl.ds(s * ZROWS, ZROWS)])
    w = c * NS + s
    pltpu.sync_copy(src_hbm.at[pl.ds(w * RPT, RPT)], srcv)
    pltpu.sync_copy(dst_hbm.at[pl.ds(w * RPT, RPT)], dstv)
    plsc.subcore_barrier()

    def g_start(k, b):
        pltpu.async_copy(hps.at[srcv.at[k]], rows.at[b], gsem[b])

    def g_wait(k, b):
        pltpu.make_async_copy(hps.at[srcv.at[k]], rows.at[b],
                              gsem[b]).wait()

    def s_start(k, b):
        pltpu.async_copy(rows.at[b], acc.at[dstv.at[k]], ssem[b], add=True)

    def s_wait(k, b):
        pltpu.make_async_copy(rows.at[b], acc.at[dstv.at[k]],
                              ssem[b]).wait()

    for b in range(LAG):
        g_start(b, b)

    def group(g, _):
        for b in range(NBUF):
            k = g * NBUF + b
            b2 = (b + LAG) % NBUF
            g_wait(k, b)
            s_start(k, b)

            @pl.when(k >= NBUF - LAG)
            def _():
                s_wait(k + LAG - NBUF, b2)

            @pl.when(k + LAG < RPT)
            def _():
                g_start(k + LAG, b2)

        return _

    lax.fori_loop(0, NG, group, None, unroll=False)
    for j in range(NBUF - LAG):
        k = RPT - (NBUF - LAG) + j
        s_wait(k, k % NBUF)
    plsc.subcore_barrier()
    pltpu.sync_copy(acc.at[pl.ds(s * ZROWS, ZROWS)],
                    out_hbm.at[c, pl.ds(s * ZROWS, ZROWS)])


_sc_agg = pl.kernel(
    _agg_body,
    out_type=jax.ShapeDtypeStruct((NC, NP, D_H), jnp.float32),
    mesh=_mesh,
    compiler_params=_sc_params,
    scratch_types=[
        pltpu.VMEM((RPT, CHUNK), jnp.int32),
        pltpu.VMEM((RPT, CHUNK), jnp.int32),
        pltpu.VMEM((NBUF, CHUNK, D_H), jnp.float32),
        pltpu.VMEM_SHARED((NP, D_H), jnp.float32),
        pltpu.VMEM_SHARED((NP, D_H), jnp.float32),
    ] + [pltpu.SemaphoreType.DMA] * (2 * NBUF),
)


_R = 1024
_GRID = NP // _R


def _tca_body(x_ref, w1_ref, h1_ref):
    h1_ref[...] = jnp.dot(x_ref[...], w1_ref[...],
                          preferred_element_type=jnp.float32)


_tca = pl.pallas_call(
    _tca_body,
    grid=(_GRID,),
    in_specs=[
        pl.BlockSpec((_R, D_IN), lambda i: (i, 0)),
        pl.BlockSpec((D_IN, D_H), lambda i: (0, 0)),
    ],
    out_specs=pl.BlockSpec((_R, D_H), lambda i: (i, 0)),
    out_shape=jax.ShapeDtypeStruct((NP, D_H), jnp.float32),
)


def _tcb_body(h1_ref, d0_ref, d1_ref, h1p_ref, dinv_ref):
    dinv = lax.rsqrt(d0_ref[0] + d1_ref[0] + 1.0)
    h1p_ref[...] = h1_ref[...] * dinv
    dinv_ref[...] = dinv


_tcb = pl.pallas_call(
    _tcb_body,
    grid=(_GRID,),
    in_specs=[
        pl.BlockSpec((_R, D_H), lambda i: (i, 0)),
        pl.BlockSpec((1, _R, D_H), lambda i: (0, i, 0)),
        pl.BlockSpec((1, _R, D_H), lambda i: (1, i, 0)),
    ],
    out_specs=[
        pl.BlockSpec((_R, D_H), lambda i: (i, 0)),
        pl.BlockSpec((_R, D_H), lambda i: (i, 0)),
    ],
    out_shape=[
        jax.ShapeDtypeStruct((NP, D_H), jnp.float32),
        jax.ShapeDtypeStruct((NP, D_H), jnp.float32),
    ],
)


def _tc2_body(s0_ref, s1_ref, h1p_ref, dinv_ref, b1_ref, w2_ref, h2p_ref):
    dinv = dinv_ref[...]
    a = dinv * (s0_ref[0] + s1_ref[0] + h1p_ref[...]) + b1_ref[...]
    r = jnp.maximum(a, 0.0)
    h2 = jnp.dot(r, w2_ref[...], preferred_element_type=jnp.float32)
    h2p_ref[...] = h2 * dinv


_tc2 = pl.pallas_call(
    _tc2_body,
    grid=(_GRID,),
    in_specs=[
        pl.BlockSpec((1, _R, D_H), lambda i: (0, i, 0)),
        pl.BlockSpec((1, _R, D_H), lambda i: (1, i, 0)),
        pl.BlockSpec((_R, D_H), lambda i: (i, 0)),
        pl.BlockSpec((_R, D_H), lambda i: (i, 0)),
        pl.BlockSpec((1, D_H), lambda i: (0, 0)),
        pl.BlockSpec((D_H, D_H), lambda i: (0, 0)),
    ],
    out_specs=pl.BlockSpec((_R, D_H), lambda i: (i, 0)),
    out_shape=jax.ShapeDtypeStruct((NP, D_H), jnp.float32),
)


def _tc3_body(s0_ref, s1_ref, h2p_ref, dinv_ref, b2_ref, batch_ref,
              wf_ref, bf_ref, out_ref, psum_ref, cnt_ref):
    i = pl.program_id(0)

    @pl.when(i == 0)
    def _init():
        psum_ref[...] = jnp.zeros_like(psum_ref)
        cnt_ref[...] = jnp.zeros_like(cnt_ref)

    dinv = dinv_ref[...]
    out2 = dinv * (s0_ref[0] + s1_ref[0] + h2p_ref[...]) + b2_ref[...]
    b = batch_ref[...].reshape(1, _R)
    valid = (batch_ref[...].reshape(_R, 1) < G)
    out2 = jnp.where(valid, out2, 0.0)
    onehot = (lax.broadcasted_iota(jnp.int32, (G, _R), 0) == b
              ).astype(jnp.float32)
    psum_ref[...] += jnp.dot(onehot, out2, preferred_element_type=jnp.float32)
    cnt_ref[...] += jnp.dot(onehot, jnp.ones((_R, D_H), jnp.float32),
                            preferred_element_type=jnp.float32)

    @pl.when(i == _GRID - 1)
    def _fin():
        pooled = psum_ref[...] / jnp.maximum(cnt_ref[...], 1.0)
        out_ref[...] = jnp.dot(pooled, wf_ref[...],
                               preferred_element_type=jnp.float32) + bf_ref[...]


_tc3 = pl.pallas_call(
    _tc3_body,
    grid=(_GRID,),
    in_specs=[
        pl.BlockSpec((1, _R, D_H), lambda i: (0, i, 0)),
        pl.BlockSpec((1, _R, D_H), lambda i: (1, i, 0)),
        pl.BlockSpec((_R, D_H), lambda i: (i, 0)),
        pl.BlockSpec((_R, D_H), lambda i: (i, 0)),
        pl.BlockSpec((1, D_H), lambda i: (0, 0)),
        pl.BlockSpec((1, 1, _R), lambda i: (i, 0, 0)),
        pl.BlockSpec((D_H, NCLS), lambda i: (0, 0)),
        pl.BlockSpec((1, NCLS), lambda i: (0, 0)),
    ],
    out_specs=pl.BlockSpec((G, NCLS), lambda i: (0, 0)),
    out_shape=jax.ShapeDtypeStruct((G, NCLS), jnp.float32),
    scratch_shapes=[
        pltpu.VMEM((G, D_H), jnp.float32),
        pltpu.VMEM((G, D_H), jnp.float32),
    ],
)


def kernel(x, edge_index, batch, W1, b1, W2, b2, Wf, bf):
    pad = jnp.full((EPAD - E,), NP - 1, dtype=jnp.int32)
    src2d = jnp.concatenate([edge_index[0], pad]).reshape(-1, CHUNK)
    dst2d = jnp.concatenate([edge_index[1], pad]).reshape(-1, CHUNK)
    batchp = jnp.concatenate(
        [batch, jnp.full((NP - N,), G, dtype=jnp.int32)]).reshape(_GRID, 1, _R)
    zeros_np = jnp.zeros((NP, D_H), jnp.float32)
    ones_chunk = jnp.ones((CHUNK, D_H), jnp.float32)

    xp = jnp.pad(x, ((0, NP - N), (0, 0)))
    h1 = _tca(xp, W1)
    deg = _sc_hist(dst2d, zeros_np, ones_chunk)
    h1p, dinv = _tcb(h1, deg, deg)
    s1 = _sc_agg(h1p, src2d, dst2d, zeros_np)
    h2p = _tc2(s1, s1, h1p, dinv, b1.reshape(1, D_H), W2)
    s2 = _sc_agg(h2p, src2d, dst2d, zeros_np)
    return _tc3(s2, s2, h2p, dinv, b2.reshape(1, D_H),
                batchp, Wf, bf.reshape(1, NCLS))


# speedup vs baseline: 54.1110x; 1.0118x over previous
import functools

import jax
import jax.numpy as jnp
from jax import lax
from jax.experimental import pallas as pl
from jax.experimental.pallas import tpu as pltpu
from jax.experimental.pallas import tpu_sc as plsc

N = 10000
E = 320000
G = 64
D_IN = 128
D_H = 16
NCLS = 10

NC = 2
NS = 16
NW = NC * NS

NP = 10240
EPT = 10240
EPAD = EPT * NW
CHUNK = 128
RPT = EPT // CHUNK
SC_ROWS = 16
NSC = RPT // SC_ROWS
ZROWS = NP // NS

_mesh = plsc.VectorSubcoreMesh(core_axis_name="c", subcore_axis_name="s")
_sc_params = pltpu.CompilerParams(use_tc_tiling_on_sc=False)


NBUF = 8
LAG = 4
NG = RPT // NBUF


def _hist_body(dst_hbm, zero_hbm, ones_hbm, out_hbm, dstv, rows, acc, *sems):
    c = lax.axis_index("c")
    s = lax.axis_index("s")
    pltpu.sync_copy(zero_hbm.at[pl.ds(s * ZROWS, ZROWS)],
                    acc.at[pl.ds(s * ZROWS, ZROWS)])
    pltpu.sync_copy(ones_hbm, rows)
    w = c * NS + s
    pltpu.sync_copy(dst_hbm.at[pl.ds(w * RPT, RPT)], dstv)
    plsc.subcore_barrier()

    def s_start(k, b):
        pltpu.async_copy(rows, acc.at[dstv.at[k]], sems[b], add=True)

    def s_wait(k, b):
        pltpu.make_async_copy(rows, acc.at[dstv.at[k]], sems[b]).wait()

    def group(g, _):
        for b in range(NBUF):
            k = g * NBUF + b

            @pl.when(k >= NBUF)
            def _():
                s_wait(k - NBUF, b)

            s_start(k, b)
        return _

    lax.fori_loop(0, NG, group, None, unroll=False)
    for b in range(NBUF):
        s_wait(RPT - NBUF + b, b)
    plsc.subcore_barrier()
    pltpu.sync_copy(acc.at[pl.ds(s * ZROWS, ZROWS)],
                    out_hbm.at[c, pl.ds(s * ZROWS, ZROWS)])


_sc_hist = pl.kernel(
    _hist_body,
    out_type=jax.ShapeDtypeStruct((NC, NP, D_H), jnp.float32),
    mesh=_mesh,
    compiler_params=_sc_params,
    scratch_types=[
        pltpu.VMEM((RPT, CHUNK), jnp.int32),
        pltpu.VMEM((CHUNK, D_H), jnp.float32),
        pltpu.VMEM_SHARED((NP, D_H), jnp.float32),
    ] + [pltpu.SemaphoreType.DMA] * NBUF,
)


def _agg_body(hp_hbm, src_hbm, dst_hbm, zero_hbm, out_hbm,
              srcv, dstv, rows, acc, hps, *sems):
    gsem = sems[:NBUF]
    ssem = sems[NBUF:]
    c = lax.axis_index("c")
    s = lax.axis_index("s")
    pltpu.sync_copy(zero_hbm.at[pl.ds(s * ZROWS, ZROWS)],
                    acc.at[pl.ds(s * ZROWS, ZROWS)])

    @pl.when(s < NS - 1)
    def _stage_full():
        pltpu.sync_copy(hp_hbm.at[pl.ds(s * ZROWS, ZROWS)],
                        hps.at[pl.ds(s * ZROWS, ZROWS)])

    @pl.when(s == NS - 1)
    def _stage_tail():
        pltpu.sync_copy(hp_hbm.at[pl.ds((NS - 1) * ZROWS,
                                        N - (NS - 1) * ZROWS)],
                        hps.at[pl.ds((NS - 1) * ZROWS,
                                     N - (NS - 1) * ZROWS)])
        pltpu.sync_copy(zero_hbm.at[pl.ds(N, NP - N)],
                        hps.at[pl.ds(N, NP - N)])

    w = c * NS + s
    pltpu.sync_copy(src_hbm.at[pl.ds(w * RPT, RPT)], srcv)
    pltpu.sync_copy(dst_hbm.at[pl.ds(w * RPT, RPT)], dstv)
    plsc.subcore_barrier()

    def g_start(k, b):
        pltpu.async_copy(hps.at[srcv.at[k]], rows.at[b], gsem[b])

    def g_wait(k, b):
        pltpu.make_async_copy(hps.at[srcv.at[k]], rows.at[b],
                              gsem[b]).wait()

    def s_start(k, b):
        pltpu.async_copy(rows.at[b], acc.at[dstv.at[k]], ssem[b], add=True)

    def s_wait(k, b):
        pltpu.make_async_copy(rows.at[b], acc.at[dstv.at[k]],
                              ssem[b]).wait()

    for b in range(LAG):
        g_start(b, b)

    def group(g, _):
        for b in range(NBUF):
            k = g * NBUF + b
            b2 = (b + LAG) % NBUF
            g_wait(k, b)
            s_start(k, b)

            @pl.when(k >= NBUF - LAG)
            def _():
                s_wait(k + LAG - NBUF, b2)

            @pl.when(k + LAG < RPT)
            def _():
                g_start(k + LAG, b2)

        return _

    lax.fori_loop(0, NG, group, None, unroll=False)
    for j in range(NBUF - LAG):
        k = RPT - (NBUF - LAG) + j
        s_wait(k, k % NBUF)
    plsc.subcore_barrier()
    pltpu.sync_copy(acc.at[pl.ds(s * ZROWS, ZROWS)],
                    out_hbm.at[c, pl.ds(s * ZROWS, ZROWS)])


_sc_agg = pl.kernel(
    _agg_body,
    out_type=jax.ShapeDtypeStruct((NC, NP, D_H), jnp.float32),
    mesh=_mesh,
    compiler_params=_sc_params,
    scratch_types=[
        pltpu.VMEM((RPT, CHUNK), jnp.int32),
        pltpu.VMEM((RPT, CHUNK), jnp.int32),
        pltpu.VMEM((NBUF, CHUNK, D_H), jnp.float32),
        pltpu.VMEM_SHARED((NP, D_H), jnp.float32),
        pltpu.VMEM_SHARED((NP, D_H), jnp.float32),
    ] + [pltpu.SemaphoreType.DMA] * (2 * NBUF),
)


_R = 1000
_GRID = N // _R


def _tca_body(x_ref, w1_ref, h1_ref):
    h1_ref[...] = jnp.dot(x_ref[...], w1_ref[...],
                          preferred_element_type=jnp.float32)


_tca = pl.pallas_call(
    _tca_body,
    grid=(_GRID,),
    in_specs=[
        pl.BlockSpec((_R, D_IN), lambda i: (i, 0)),
        pl.BlockSpec((D_IN, D_H), lambda i: (0, 0)),
    ],
    out_specs=pl.BlockSpec((_R, D_H), lambda i: (i, 0)),
    out_shape=jax.ShapeDtypeStruct((N, D_H), jnp.float32),
)


def _tcb_body(h1_ref, d0_ref, d1_ref, h1p_ref, dinv_ref):
    dinv = lax.rsqrt(d0_ref[0] + d1_ref[0] + 1.0)
    h1p_ref[...] = h1_ref[...] * dinv
    dinv_ref[...] = dinv


_tcb = pl.pallas_call(
    _tcb_body,
    grid=(_GRID,),
    in_specs=[
        pl.BlockSpec((_R, D_H), lambda i: (i, 0)),
        pl.BlockSpec((1, _R, D_H), lambda i: (0, i, 0)),
        pl.BlockSpec((1, _R, D_H), lambda i: (1, i, 0)),
    ],
    out_specs=[
        pl.BlockSpec((_R, D_H), lambda i: (i, 0)),
        pl.BlockSpec((_R, D_H), lambda i: (i, 0)),
    ],
    out_shape=[
        jax.ShapeDtypeStruct((N, D_H), jnp.float32),
        jax.ShapeDtypeStruct((N, D_H), jnp.float32),
    ],
)


def _tc2_body(s0_ref, s1_ref, h1p_ref, dinv_ref, b1_ref, w2_ref, h2p_ref):
    dinv = dinv_ref[...]
    a = dinv * (s0_ref[0] + s1_ref[0] + h1p_ref[...]) + b1_ref[...]
    r = jnp.maximum(a, 0.0)
    h2 = jnp.dot(r, w2_ref[...], preferred_element_type=jnp.float32)
    h2p_ref[...] = h2 * dinv


_tc2 = pl.pallas_call(
    _tc2_body,
    grid=(_GRID,),
    in_specs=[
        pl.BlockSpec((1, _R, D_H), lambda i: (0, i, 0)),
        pl.BlockSpec((1, _R, D_H), lambda i: (1, i, 0)),
        pl.BlockSpec((_R, D_H), lambda i: (i, 0)),
        pl.BlockSpec((_R, D_H), lambda i: (i, 0)),
        pl.BlockSpec((1, D_H), lambda i: (0, 0)),
        pl.BlockSpec((D_H, D_H), lambda i: (0, 0)),
    ],
    out_specs=pl.BlockSpec((_R, D_H), lambda i: (i, 0)),
    out_shape=jax.ShapeDtypeStruct((N, D_H), jnp.float32),
)


def _tc3_body(s0_ref, s1_ref, h2p_ref, dinv_ref, b2_ref, batch_ref,
              wf_ref, bf_ref, out_ref, psum_ref, cnt_ref):
    i = pl.program_id(0)

    @pl.when(i == 0)
    def _init():
        psum_ref[...] = jnp.zeros_like(psum_ref)
        cnt_ref[...] = jnp.zeros_like(cnt_ref)

    dinv = dinv_ref[...]
    out2 = dinv * (s0_ref[0] + s1_ref[0] + h2p_ref[...]) + b2_ref[...]
    b = batch_ref[...].reshape(1, _R)
    onehot = (lax.broadcasted_iota(jnp.int32, (G, _R), 0) == b
              ).astype(jnp.float32)
    psum_ref[...] += jnp.dot(onehot, out2, preferred_element_type=jnp.float32)
    cnt_ref[...] += jnp.dot(onehot, jnp.ones((_R, D_H), jnp.float32),
                            preferred_element_type=jnp.float32)

    @pl.when(i == _GRID - 1)
    def _fin():
        pooled = psum_ref[...] / jnp.maximum(cnt_ref[...], 1.0)
        out_ref[...] = jnp.dot(pooled, wf_ref[...],
                               preferred_element_type=jnp.float32) + bf_ref[...]


_tc3 = pl.pallas_call(
    _tc3_body,
    grid=(_GRID,),
    in_specs=[
        pl.BlockSpec((1, _R, D_H), lambda i: (0, i, 0)),
        pl.BlockSpec((1, _R, D_H), lambda i: (1, i, 0)),
        pl.BlockSpec((_R, D_H), lambda i: (i, 0)),
        pl.BlockSpec((_R, D_H), lambda i: (i, 0)),
        pl.BlockSpec((1, D_H), lambda i: (0, 0)),
        pl.BlockSpec((1, 1, _R), lambda i: (i, 0, 0)),
        pl.BlockSpec((D_H, NCLS), lambda i: (0, 0)),
        pl.BlockSpec((1, NCLS), lambda i: (0, 0)),
    ],
    out_specs=pl.BlockSpec((G, NCLS), lambda i: (0, 0)),
    out_shape=jax.ShapeDtypeStruct((G, NCLS), jnp.float32),
    scratch_shapes=[
        pltpu.VMEM((G, D_H), jnp.float32),
        pltpu.VMEM((G, D_H), jnp.float32),
    ],
)


def kernel(x, edge_index, batch, W1, b1, W2, b2, Wf, bf):
    pad = jnp.full((EPAD - E,), NP - 1, dtype=jnp.int32)
    src2d = jnp.concatenate([edge_index[0], pad]).reshape(-1, CHUNK)
    dst2d = jnp.concatenate([edge_index[1], pad]).reshape(-1, CHUNK)
    batchp = batch.reshape(_GRID, 1, _R)
    zeros_np = jnp.zeros((NP, D_H), jnp.float32)
    ones_chunk = jnp.ones((CHUNK, D_H), jnp.float32)

    h1 = _tca(x, W1)
    deg = _sc_hist(dst2d, zeros_np, ones_chunk)
    h1p, dinv = _tcb(h1, deg, deg)
    s1 = _sc_agg(h1p, src2d, dst2d, zeros_np)
    h2p = _tc2(s1, s1, h1p, dinv, b1.reshape(1, D_H), W2)
    s2 = _sc_agg(h2p, src2d, dst2d, zeros_np)
    return _tc3(s2, s2, h2p, dinv, b2.reshape(1, D_H),
                batchp, Wf, bf.reshape(1, NCLS))
